# Initial kernel scaffold; baseline (speedup 1.0000x reference)
import jax
import jax.numpy as jnp
from jax.experimental import pallas as pl


def kernel(phi_im, phi_cli, t, traumatic):
    raise NotImplementedError("write your pallas kernel here")


# speedup vs baseline: 1.9051x; 1.9051x over previous
import jax
import jax.numpy as jnp
from jax.experimental import pallas as pl
from jax.experimental.pallas import tpu as pltpu

N = 4096
D = 128
BLK = 512
NBLK = N // BLK
BIG = 1000.0


def _normalize(x):
    nrm = jnp.sqrt(jnp.sum(x * x, axis=1, keepdims=True))
    return x / jnp.clip(nrm, 1e-12)


def _fused_body(im_ref, cli_ref, tcol_r_ref, tr_r_ref, tcol_c_ref, tr_c_ref,
                out_ref):
    i = pl.program_id(0)

    @pl.when(i == 0)
    def _init():
        for k in range(6):
            out_ref[k] = BIG

    imn = _normalize(im_ref[...])
    clin = _normalize(cli_ref[...])

    tcol_r = tcol_r_ref[...]
    tr_r = tr_r_ref[...]
    cm1 = tcol_r == 1
    cm2 = tcol_r == 2

    tcol_c = tcol_c_ref[...]
    tr_c = tr_c_ref[...]
    rm0 = jnp.logical_and(tcol_c == 0, tr_c == 1)
    rm1 = tcol_c == 1

    rows_im = _normalize(im_ref[pl.ds(i * BLK, BLK), :])
    rows_cli = _normalize(cli_ref[pl.ds(i * BLK, BLK), :])

    dn = (((1,), (1,)), ((), ()))

    def masked_mins(rows_n, cols_n):
        sim = jax.lax.dot_general(rows_n, cols_n, dn,
                                  preferred_element_type=jnp.float32)
        c1 = jnp.min(jnp.where(cm1, sim, BIG), axis=1, keepdims=True)
        c2 = jnp.min(jnp.where(cm2, sim, BIG), axis=1, keepdims=True)
        s01 = jnp.min(jnp.where(rm0, c1, BIG))
        s02 = jnp.min(jnp.where(rm0, c2, BIG))
        s12 = jnp.min(jnp.where(rm1, c2, BIG))
        return s01, s02, s12

    vals = masked_mins(rows_im, imn) + masked_mins(rows_cli, clin)
    for k in range(6):
        out_ref[k] = jnp.minimum(out_ref[k], vals[k])


def kernel(phi_im, phi_cli, t, traumatic):
    tcol = t[:, 1]
    tcol_row = tcol[None, :].astype(jnp.int32)
    tr_row = traumatic[None, :].astype(jnp.int32)
    tcol_col = tcol[:, None].astype(jnp.int32)
    tr_col = traumatic[:, None].astype(jnp.int32)

    full = lambda shape: pl.BlockSpec(shape, lambda i: (0, 0))
    rowblk = pl.BlockSpec((BLK, 1), lambda i: (i, 0))

    out = pl.pallas_call(
        _fused_body,
        grid=(NBLK,),
        in_specs=[
            full((N, D)),
            full((N, D)),
            full((1, N)),
            full((1, N)),
            rowblk,
            rowblk,
        ],
        out_specs=pl.BlockSpec(memory_space=pltpu.SMEM),
        out_shape=jax.ShapeDtypeStruct((6,), jnp.float32),
        compiler_params=pltpu.CompilerParams(
            dimension_semantics=("arbitrary",),
        ),
    )(phi_im, phi_cli, tcol_row, tr_row, tcol_col, tr_col)
    return out
